# Initial kernel scaffold; baseline (speedup 1.0000x reference)
import jax
import jax.numpy as jnp
from jax.experimental import pallas as pl


def kernel(z, codebook, W, proj_bias):
    raise NotImplementedError("write your pallas kernel here")


# speedup vs baseline: 3.7324x; 3.7324x over previous
import functools

import jax
import jax.numpy as jnp
from jax import lax
from jax.experimental import pallas as pl
from jax.experimental.pallas import tpu as pltpu
from jax.experimental.pallas import tpu_sc as plsc

K = 8192
D = 32
N_TOK = 8 * 4096
TB = 256
NTB = N_TOK // TB

_NC = 2
_NS = 16
NW = _NC * _NS
TPW = N_TOK // NW
NCH = TPW // 128


def _search_body(z_ref, cb_ref, w_ref, b_ref, idx_ref, cbp_ref,
                 negt_ref, emb_ref):
    pid = pl.program_id(0)

    @pl.when(pid == 0)
    def _():
        cbp = jnp.dot(cb_ref[...], w_ref[...],
                      preferred_element_type=jnp.float32) + b_ref[...]
        cbp_ref[...] = cbp
        negt_ref[...] = cbp * -2.0
        emb_ref[...] = jnp.sum(cbp * cbp, axis=1)[None, :]

    z = z_ref[...]
    dots2 = lax.dot_general(z, negt_ref[...], (((1,), (1,)), ((), ())),
                            preferred_element_type=jnp.float32)
    scores = emb_ref[...] + dots2
    idx_ref[0, 0, :] = jnp.argmin(scores, axis=1).astype(jnp.int32)


_search = pl.pallas_call(
    _search_body,
    grid=(NTB,),
    in_specs=[
        pl.BlockSpec((TB, D), lambda i: (i, 0)),
        pl.BlockSpec((K, D), lambda i: (0, 0)),
        pl.BlockSpec((D, D), lambda i: (0, 0)),
        pl.BlockSpec((1, D), lambda i: (0, 0)),
    ],
    out_specs=[
        pl.BlockSpec((1, 1, TB), lambda i: (i, 0, 0)),
        pl.BlockSpec((K, D), lambda i: (0, 0)),
    ],
    out_shape=[
        jax.ShapeDtypeStruct((NTB, 1, TB), jnp.int32),
        jax.ShapeDtypeStruct((K, D), jnp.float32),
    ],
    scratch_shapes=[pltpu.VMEM((K, D), jnp.float32),
                    pltpu.VMEM((1, K), jnp.float32)],
)


@functools.cache
def _make_sc_gather_counts():
    @functools.partial(
        pl.kernel,
        mesh=plsc.VectorSubcoreMesh(core_axis_name="c", subcore_axis_name="s"),
        compiler_params=pltpu.CompilerParams(use_tc_tiling_on_sc=False),
        out_type=[
            jax.ShapeDtypeStruct((N_TOK, D), jnp.float32),
            jax.ShapeDtypeStruct((_NC, K), jnp.float32),
        ],
        scratch_types=[
            pltpu.VMEM((NCH, 128), jnp.int32),
            pltpu.VMEM((TPW, D), jnp.float32),
            pltpu.VMEM((128,), jnp.float32),
            pltpu.VMEM((K // _NS,), jnp.float32),
            pltpu.VMEM_SHARED((K,), jnp.float32),
            pltpu.VMEM_SHARED((K, D), jnp.float32),
            pltpu.SemaphoreType.DMA,
        ],
    )
    def _sc_gather_counts(idx_hbm, table_hbm, zq_out, counts_out,
                          idx_v, rows_v, ones_v, zeros_v, hist, table_sh, sem):
        cid = lax.axis_index("c")
        sid = lax.axis_index("s")
        wid = sid * _NC + cid
        zslice = K // _NS

        for i in range(128 // 16):
            ones_v[pl.ds(i * 16, 16)] = jnp.ones((16,), jnp.float32)
        for i in range(zslice // 16):
            zeros_v[pl.ds(i * 16, 16)] = jnp.zeros((16,), jnp.float32)

        pltpu.sync_copy(idx_hbm.at[pl.ds(wid * NCH, NCH)], idx_v)

        pltpu.sync_copy(table_hbm.at[pl.ds(sid * zslice, zslice)],
                        table_sh.at[pl.ds(sid * zslice, zslice)])

        pltpu.sync_copy(zeros_v, hist.at[pl.ds(sid * zslice, zslice)])
        plsc.subcore_barrier()

        cps = [pltpu.async_copy(table_sh.at[idx_v.at[j]],
                                rows_v.at[pl.ds(j * 128, 128)], sem)
               for j in range(NCH)]
        for c in cps:
            c.wait()
        pltpu.sync_copy(rows_v, zq_out.at[pl.ds(wid * TPW, TPW)])

        for j in range(NCH):
            pltpu.sync_copy(ones_v, hist.at[idx_v.at[j]], add=True)
        plsc.subcore_barrier()

        @pl.when(sid == 0)
        def _():
            pltpu.sync_copy(hist, counts_out.at[cid])

    return _sc_gather_counts


def _stats_body(cp_ref, counts_ref, total_ref, avg_ref, ppl_ref, usage_ref):
    c = cp_ref[0] + cp_ref[1]
    counts_ref[...] = c
    total = jnp.maximum(jnp.sum(c), 1.0)
    avg = c / total
    avg_ref[...] = avg
    safe = jnp.where(avg > 0, avg, 1.0)
    ppl_ref[...] = jnp.exp(-jnp.sum(avg * jnp.log(safe + 1e-10))).reshape(1, 1)
    usage_ref[...] = jnp.mean((c > 0).astype(jnp.float32)).reshape(1, 1)
    total_ref[...] = total.reshape(1, 1)


_stats = pl.pallas_call(
    _stats_body,
    out_shape=[
        jax.ShapeDtypeStruct((K // 128, 128), jnp.float32),
        jax.ShapeDtypeStruct((1, 1), jnp.float32),
        jax.ShapeDtypeStruct((K // 128, 128), jnp.float32),
        jax.ShapeDtypeStruct((1, 1), jnp.float32),
        jax.ShapeDtypeStruct((1, 1), jnp.float32),
    ],
)


def kernel(z, codebook, W, proj_bias):
    b, t, c = z.shape
    z_flat = z.reshape(b * t, c)
    idx3, cb_proj = _search(z_flat, codebook, W, proj_bias.reshape(1, D))
    idx_rows = idx3.reshape(N_TOK // 128, 128)
    zq_flat, count_partials = _make_sc_gather_counts()(idx_rows, cb_proj)
    counts2d, total, avg2d, ppl, usage = _stats(
        count_partials.reshape(_NC, K // 128, 128))
    z_q = zq_flat.reshape(b, t, c)
    indices_bt = idx3.reshape(b, t)
    counts = counts2d.reshape(K)
    avg_probs = avg2d.reshape(K)
    commit_loss = jnp.zeros((), jnp.float32)
    return (z_q, indices_bt, commit_loss, ppl.reshape(()), usage.reshape(()),
            counts, total.reshape(()), avg_probs)


# speedup vs baseline: 3.8487x; 1.0312x over previous
import functools

import jax
import jax.numpy as jnp
from jax import lax
from jax.experimental import pallas as pl
from jax.experimental.pallas import tpu as pltpu
from jax.experimental.pallas import tpu_sc as plsc

K = 8192
D = 32
N_TOK = 8 * 4096
TB = 512
NTB = N_TOK // TB
JC = 256

_NC = 2
_NS = 16
NW = _NC * _NS
TPW = N_TOK // NW
NCH = TPW // 128


def _search_body(z_ref, cb_ref, w_ref, b_ref, idx_ref, cbp_ref,
                 negt_ref, emb_ref):
    pid = pl.program_id(0)

    @pl.when(pid == 0)
    def _():
        cbp = jnp.dot(cb_ref[...], w_ref[...],
                      preferred_element_type=jnp.float32) + b_ref[...]
        cbp_ref[...] = cbp
        negt_ref[...] = cbp * -2.0
        emb_ref[...] = jnp.sum(cbp * cbp, axis=1)[None, :]

    z = z_ref[...]
    best = jnp.full((TB, 128), jnp.inf, jnp.float32)
    bestj = jnp.zeros((TB, 128), jnp.int32)
    for jb in range(K // JC):
        d2 = lax.dot_general(z, negt_ref[jb * JC:(jb + 1) * JC, :],
                             (((1,), (1,)), ((), ())),
                             preferred_element_type=jnp.float32)
        s = emb_ref[0:1, jb * JC:(jb + 1) * JC] + d2
        for c in range(JC // 128):
            tile = s[:, c * 128:(c + 1) * 128]
            j = jb * (JC // 128) + c
            m = tile < best
            best = jnp.where(m, tile, best)
            bestj = jnp.where(m, j, bestj)
    lane = lax.broadcasted_iota(jnp.int32, (TB, 128), 1)
    gk = bestj * 128 + lane
    mv = jnp.min(best, axis=1, keepdims=True)
    cand = jnp.where(best == mv, gk, jnp.int32(2 * K))
    idx_ref[0, 0, :] = jnp.min(cand, axis=1).astype(jnp.int32)


_search = pl.pallas_call(
    _search_body,
    grid=(NTB,),
    in_specs=[
        pl.BlockSpec((TB, D), lambda i: (i, 0)),
        pl.BlockSpec((K, D), lambda i: (0, 0)),
        pl.BlockSpec((D, D), lambda i: (0, 0)),
        pl.BlockSpec((1, D), lambda i: (0, 0)),
    ],
    out_specs=[
        pl.BlockSpec((1, 1, TB), lambda i: (i, 0, 0)),
        pl.BlockSpec((K, D), lambda i: (0, 0)),
    ],
    out_shape=[
        jax.ShapeDtypeStruct((NTB, 1, TB), jnp.int32),
        jax.ShapeDtypeStruct((K, D), jnp.float32),
    ],
    scratch_shapes=[pltpu.VMEM((K, D), jnp.float32),
                    pltpu.VMEM((1, K), jnp.float32)],
)


@functools.cache
def _make_sc_gather_counts():
    @functools.partial(
        pl.kernel,
        mesh=plsc.VectorSubcoreMesh(core_axis_name="c", subcore_axis_name="s"),
        compiler_params=pltpu.CompilerParams(use_tc_tiling_on_sc=False),
        out_type=[
            jax.ShapeDtypeStruct((N_TOK, D), jnp.float32),
            jax.ShapeDtypeStruct((_NC, K), jnp.float32),
        ],
        scratch_types=[
            pltpu.VMEM((NCH, 128), jnp.int32),
            pltpu.VMEM((TPW, D), jnp.float32),
            pltpu.VMEM((128,), jnp.float32),
            pltpu.VMEM((K // _NS,), jnp.float32),
            pltpu.VMEM_SHARED((K,), jnp.float32),
            pltpu.VMEM_SHARED((K, D), jnp.float32),
            pltpu.SemaphoreType.DMA,
        ],
    )
    def _sc_gather_counts(idx_hbm, table_hbm, zq_out, counts_out,
                          idx_v, rows_v, ones_v, zeros_v, hist, table_sh, sem):
        cid = lax.axis_index("c")
        sid = lax.axis_index("s")
        wid = sid * _NC + cid
        zslice = K // _NS

        for i in range(128 // 16):
            ones_v[pl.ds(i * 16, 16)] = jnp.ones((16,), jnp.float32)
        for i in range(zslice // 16):
            zeros_v[pl.ds(i * 16, 16)] = jnp.zeros((16,), jnp.float32)

        pltpu.sync_copy(idx_hbm.at[pl.ds(wid * NCH, NCH)], idx_v)

        pltpu.sync_copy(table_hbm.at[pl.ds(sid * zslice, zslice)],
                        table_sh.at[pl.ds(sid * zslice, zslice)])

        pltpu.sync_copy(zeros_v, hist.at[pl.ds(sid * zslice, zslice)])
        plsc.subcore_barrier()

        cps = [pltpu.async_copy(table_sh.at[idx_v.at[j]],
                                rows_v.at[pl.ds(j * 128, 128)], sem)
               for j in range(NCH)]
        for c in cps:
            c.wait()
        pltpu.sync_copy(rows_v, zq_out.at[pl.ds(wid * TPW, TPW)])

        for j in range(NCH):
            pltpu.sync_copy(ones_v, hist.at[idx_v.at[j]], add=True)
        plsc.subcore_barrier()

        @pl.when(sid == 0)
        def _():
            pltpu.sync_copy(hist, counts_out.at[cid])

    return _sc_gather_counts


def _stats_body(cp_ref, counts_ref, total_ref, avg_ref, ppl_ref, usage_ref):
    c = cp_ref[0] + cp_ref[1]
    counts_ref[...] = c
    total = jnp.maximum(jnp.sum(c), 1.0)
    avg = c / total
    avg_ref[...] = avg
    safe = jnp.where(avg > 0, avg, 1.0)
    ppl_ref[...] = jnp.exp(-jnp.sum(avg * jnp.log(safe + 1e-10))).reshape(1, 1)
    usage_ref[...] = jnp.mean((c > 0).astype(jnp.float32)).reshape(1, 1)
    total_ref[...] = total.reshape(1, 1)


_stats = pl.pallas_call(
    _stats_body,
    out_shape=[
        jax.ShapeDtypeStruct((K // 128, 128), jnp.float32),
        jax.ShapeDtypeStruct((1, 1), jnp.float32),
        jax.ShapeDtypeStruct((K // 128, 128), jnp.float32),
        jax.ShapeDtypeStruct((1, 1), jnp.float32),
        jax.ShapeDtypeStruct((1, 1), jnp.float32),
    ],
)


def kernel(z, codebook, W, proj_bias):
    b, t, c = z.shape
    z_flat = z.reshape(b * t, c)
    idx3, cb_proj = _search(z_flat, codebook, W, proj_bias.reshape(1, D))
    idx_rows = idx3.reshape(N_TOK // 128, 128)
    zq_flat, count_partials = _make_sc_gather_counts()(idx_rows, cb_proj)
    counts2d, total, avg2d, ppl, usage = _stats(
        count_partials.reshape(_NC, K // 128, 128))
    z_q = zq_flat.reshape(b, t, c)
    indices_bt = idx3.reshape(b, t)
    counts = counts2d.reshape(K)
    avg_probs = avg2d.reshape(K)
    commit_loss = jnp.zeros((), jnp.float32)
    return (z_q, indices_bt, commit_loss, ppl.reshape(()), usage.reshape(()),
            counts, total.reshape(()), avg_probs)
